# baseline (speedup 1.0000x reference)
import functools

import jax
import jax.numpy as jnp
from jax.experimental import pallas as pl
from jax.experimental.pallas import tpu as pltpu

_EPS = 1e-5


def _pick_tile(n, cap=512):
    best = 1
    for t in range(8, min(n, cap) + 1, 8):
        if n % t == 0:
            best = t
    return best if n % best == 0 and best > 1 else n


def _bn_relu_val(x, m_ref, v_ref, g_ref, be_ref):
    bn = (x - m_ref[...]) / jnp.sqrt(v_ref[...] + _EPS) * g_ref[...] + be_ref[...]
    return jnp.maximum(bn, 0.0)


def _chunk_pool(adj_ref, x_ref, kc):
    n = adj_ref.shape[1]
    acc = None
    for j in range(0, n, kc):
        a = adj_ref[:, j:j + kc].astype(jnp.bfloat16)
        p = jnp.dot(a, x_ref[j:j + kc, :], preferred_element_type=jnp.float32)
        acc = p if acc is None else acc + p
    return acc


def _pool_mm_kernel(adj_ref, x_ref, w_ref, b_ref, y_ref, sum_ref, s1,
                    *, nsteps, kc):
    i = pl.program_id(0)

    @pl.when(i == 0)
    def _init():
        s1[...] = jnp.zeros_like(s1)

    pooled = _chunk_pool(adj_ref, x_ref, kc)
    y = jnp.dot(pooled.astype(jnp.bfloat16), w_ref[...],
                preferred_element_type=jnp.float32) + b_ref[...]
    y_ref[...] = y
    s1[...] += jnp.sum(y, axis=0, keepdims=True)

    @pl.when(i == nsteps - 1)
    def _fini():
        sum_ref[...] = s1[...]


def _bn_mm_kernel(y_ref, m_ref, v_ref, g_ref, be_ref, w_ref, b_ref,
                  z_ref, sum_ref, s1, *, nsteps):
    i = pl.program_id(0)

    @pl.when(i == 0)
    def _init():
        s1[...] = jnp.zeros_like(s1)

    h = _bn_relu_val(y_ref[...], m_ref, v_ref, g_ref, be_ref)
    z = jnp.dot(h.astype(jnp.bfloat16), w_ref[...],
                preferred_element_type=jnp.float32) + b_ref[...]
    z_ref[...] = z
    s1[...] += jnp.sum(z, axis=0, keepdims=True)

    @pl.when(i == nsteps - 1)
    def _fini():
        sum_ref[...] = s1[...]


def _col_var_kernel(y_ref, m_ref, var_ref, s, *, nsteps):
    i = pl.program_id(0)

    @pl.when(i == 0)
    def _init():
        s[...] = jnp.zeros_like(s)

    d = y_ref[...] - m_ref[...]
    s[...] += jnp.sum(d * d, axis=0, keepdims=True)

    @pl.when(i == nsteps - 1)
    def _fini():
        var_ref[...] = s[...]


def _col_var(y, m, tm):
    n, h = y.shape
    nsteps = n // tm
    return pl.pallas_call(
        functools.partial(_col_var_kernel, nsteps=nsteps),
        grid=(nsteps,),
        in_specs=[
            pl.BlockSpec((tm, h), lambda i: (i, 0)),
            pl.BlockSpec(m.shape, lambda i: (0, 0)),
        ],
        out_specs=pl.BlockSpec((1, h), lambda i: (0, 0)),
        out_shape=jax.ShapeDtypeStruct((1, h), jnp.float32),
        scratch_shapes=[pltpu.VMEM((1, h), jnp.float32)],
    )(y, m)


def _bn_relu_kernel(z_ref, m_ref, v_ref, g_ref, be_ref, out_ref):
    out_ref[...] = _bn_relu_val(
        z_ref[...], m_ref, v_ref, g_ref, be_ref).astype(out_ref.dtype)


def _mean_var(y, s1, n, tm):
    mean = (s1 / n).reshape(1, -1)
    var = _col_var(y, mean, tm) / n
    return mean, var


def _pool_mm(adj, x, w, b, tm, kc):
    n = adj.shape[0]
    h = w.shape[1]
    nsteps = n // tm
    return pl.pallas_call(
        functools.partial(_pool_mm_kernel, nsteps=nsteps, kc=kc),
        grid=(nsteps,),
        in_specs=[
            pl.BlockSpec((tm, n), lambda i: (i, 0)),
            pl.BlockSpec(x.shape, lambda i: (0, 0)),
            pl.BlockSpec(w.shape, lambda i: (0, 0)),
            pl.BlockSpec(b.shape, lambda i: (0, 0)),
        ],
        out_specs=[
            pl.BlockSpec((tm, h), lambda i: (i, 0)),
            pl.BlockSpec((1, h), lambda i: (0, 0)),
        ],
        out_shape=[
            jax.ShapeDtypeStruct((n, h), jnp.float32),
            jax.ShapeDtypeStruct((1, h), jnp.float32),
        ],
        scratch_shapes=[pltpu.VMEM((1, h), jnp.float32)],
    )(adj, x, w, b)


def _bn_mm(y, m, v, g, be, w, b, tm):
    n = y.shape[0]
    h = w.shape[1]
    nsteps = n // tm
    return pl.pallas_call(
        functools.partial(_bn_mm_kernel, nsteps=nsteps),
        grid=(nsteps,),
        in_specs=[
            pl.BlockSpec((tm, y.shape[1]), lambda i: (i, 0)),
            pl.BlockSpec(m.shape, lambda i: (0, 0)),
            pl.BlockSpec(v.shape, lambda i: (0, 0)),
            pl.BlockSpec(g.shape, lambda i: (0, 0)),
            pl.BlockSpec(be.shape, lambda i: (0, 0)),
            pl.BlockSpec(w.shape, lambda i: (0, 0)),
            pl.BlockSpec(b.shape, lambda i: (0, 0)),
        ],
        out_specs=[
            pl.BlockSpec((tm, h), lambda i: (i, 0)),
            pl.BlockSpec((1, h), lambda i: (0, 0)),
        ],
        out_shape=[
            jax.ShapeDtypeStruct((n, h), jnp.float32),
            jax.ShapeDtypeStruct((1, h), jnp.float32),
        ],
        scratch_shapes=[pltpu.VMEM((1, h), jnp.float32)],
    )(y, m, v, g, be, w, b)


def _bn_relu(z, m, v, g, be, tm, out_dtype=jnp.float32):
    n, h = z.shape
    nsteps = n // tm
    return pl.pallas_call(
        _bn_relu_kernel,
        grid=(nsteps,),
        in_specs=[
            pl.BlockSpec((tm, h), lambda i: (i, 0)),
            pl.BlockSpec(m.shape, lambda i: (0, 0)),
            pl.BlockSpec(v.shape, lambda i: (0, 0)),
            pl.BlockSpec(g.shape, lambda i: (0, 0)),
            pl.BlockSpec(be.shape, lambda i: (0, 0)),
        ],
        out_specs=pl.BlockSpec((tm, h), lambda i: (i, 0)),
        out_shape=jax.ShapeDtypeStruct((n, h), out_dtype),
    )(z, m, v, g, be)


def kernel(Adj, feats, W0_1, b0_1, g0_1, be0_1, W0_2, b0_2, bn0_g, bn0_b,
           W1_1, b1_1, g1_1, be1_1, W1_2, b1_2, bn1_g, bn1_b):
    n = Adj.shape[0]
    tm_big = _pick_tile(n, 1000)
    tm_small = _pick_tile(n, 2048)
    kc = n

    x0 = feats.astype(jnp.bfloat16)
    r2 = lambda a: a.reshape(1, -1)

    adj_b = Adj.astype(jnp.bfloat16)
    y0, s1 = _pool_mm(adj_b, x0, W0_1, r2(b0_1), tm_big, kc)
    m, v = _mean_var(y0, s1, n, tm_small)
    z0, s1 = _bn_mm(y0, m, v, r2(g0_1), r2(be0_1), W0_2, r2(b0_2), tm_small)
    m, v = _mean_var(z0, s1, n, tm_small)
    h0 = _bn_relu(z0, m, v, r2(bn0_g), r2(bn0_b), tm_small,
                  out_dtype=jnp.bfloat16)

    y1, s1 = _pool_mm(adj_b, h0, W1_1, r2(b1_1), tm_big, kc)
    m, v = _mean_var(y1, s1, n, tm_small)
    z1, s1 = _bn_mm(y1, m, v, r2(g1_1), r2(be1_1), W1_2, r2(b1_2), tm_small)
    m, v = _mean_var(z1, s1, n, tm_small)

    return _bn_relu(z1, m, v, r2(bn1_g), r2(bn1_b), tm_small)


# speedup vs baseline: 1.2917x; 1.2917x over previous
import functools

import jax
import jax.numpy as jnp
from jax.experimental import pallas as pl
from jax.experimental.pallas import tpu as pltpu

_EPS = 1e-5


def _pick_tile(n, cap=512):
    best = 1
    for t in range(8, min(n, cap) + 1, 8):
        if n % t == 0:
            best = t
    return best if n % best == 0 and best > 1 else n


def _bn_relu_val(x, m_ref, v_ref, g_ref, be_ref):
    bn = (x - m_ref[...]) / jnp.sqrt(v_ref[...] + _EPS) * g_ref[...] + be_ref[...]
    return jnp.maximum(bn, 0.0)


def _chunk_pool(adj_ref, x_ref, kc):
    n = adj_ref.shape[1]
    acc = None
    for j in range(0, n, kc):
        a = adj_ref[:, j:j + kc].astype(jnp.bfloat16)
        p = jnp.dot(a, x_ref[j:j + kc, :], preferred_element_type=jnp.float32)
        acc = p if acc is None else acc + p
    return acc


def _pool_mm_kernel(adj_ref, x_ref, w_ref, b_ref, y_ref, sum_ref, s1,
                    *, nsteps, kc):
    i = pl.program_id(0)

    @pl.when(i == 0)
    def _init():
        s1[...] = jnp.zeros_like(s1)

    pooled = _chunk_pool(adj_ref, x_ref, kc)
    y = jnp.dot(pooled.astype(jnp.bfloat16), w_ref[...],
                preferred_element_type=jnp.float32) + b_ref[...]
    y_ref[...] = y
    s1[...] += jnp.sum(y, axis=0, keepdims=True)

    @pl.when(i == nsteps - 1)
    def _fini():
        sum_ref[...] = s1[...]


def _bn_mm_kernel(y_ref, m_ref, v_ref, g_ref, be_ref, w_ref, b_ref,
                  z_ref, sum_ref, s1, *, nsteps):
    i = pl.program_id(0)

    @pl.when(i == 0)
    def _init():
        s1[...] = jnp.zeros_like(s1)

    h = _bn_relu_val(y_ref[...], m_ref, v_ref, g_ref, be_ref)
    z = jnp.dot(h.astype(jnp.bfloat16), w_ref[...],
                preferred_element_type=jnp.float32) + b_ref[...]
    z_ref[...] = z
    s1[...] += jnp.sum(z, axis=0, keepdims=True)

    @pl.when(i == nsteps - 1)
    def _fini():
        sum_ref[...] = s1[...]


def _col_var_kernel(y_ref, m_ref, var_ref, s, *, nsteps):
    i = pl.program_id(0)

    @pl.when(i == 0)
    def _init():
        s[...] = jnp.zeros_like(s)

    d = y_ref[...] - m_ref[...]
    s[...] += jnp.sum(d * d, axis=0, keepdims=True)

    @pl.when(i == nsteps - 1)
    def _fini():
        var_ref[...] = s[...]


def _col_var(y, m, tm):
    n, h = y.shape
    nsteps = n // tm
    return pl.pallas_call(
        functools.partial(_col_var_kernel, nsteps=nsteps),
        grid=(nsteps,),
        in_specs=[
            pl.BlockSpec((tm, h), lambda i: (i, 0)),
            pl.BlockSpec(m.shape, lambda i: (0, 0)),
        ],
        out_specs=pl.BlockSpec((1, h), lambda i: (0, 0)),
        out_shape=jax.ShapeDtypeStruct((1, h), jnp.float32),
        scratch_shapes=[pltpu.VMEM((1, h), jnp.float32)],
    )(y, m)


def _bn_relu_kernel(z_ref, m_ref, v_ref, g_ref, be_ref, out_ref):
    out_ref[...] = _bn_relu_val(
        z_ref[...], m_ref, v_ref, g_ref, be_ref).astype(out_ref.dtype)


def _mean_var(y, s1, n, tm):
    mean = (s1 / n).reshape(1, -1)
    var = _col_var(y, mean, tm) / n
    return mean, var


def _pool_mm(adj, x, w, b, tm, kc):
    n = adj.shape[0]
    h = w.shape[1]
    nsteps = n // tm
    return pl.pallas_call(
        functools.partial(_pool_mm_kernel, nsteps=nsteps, kc=kc),
        grid=(nsteps,),
        in_specs=[
            pl.BlockSpec((tm, n), lambda i: (i, 0)),
            pl.BlockSpec(x.shape, lambda i: (0, 0)),
            pl.BlockSpec(w.shape, lambda i: (0, 0)),
            pl.BlockSpec(b.shape, lambda i: (0, 0)),
        ],
        out_specs=[
            pl.BlockSpec((tm, h), lambda i: (i, 0)),
            pl.BlockSpec((1, h), lambda i: (0, 0)),
        ],
        out_shape=[
            jax.ShapeDtypeStruct((n, h), jnp.float32),
            jax.ShapeDtypeStruct((1, h), jnp.float32),
        ],
        scratch_shapes=[pltpu.VMEM((1, h), jnp.float32)],
    )(adj, x, w, b)


def _bn_mm(y, m, v, g, be, w, b, tm):
    n = y.shape[0]
    h = w.shape[1]
    nsteps = n // tm
    return pl.pallas_call(
        functools.partial(_bn_mm_kernel, nsteps=nsteps),
        grid=(nsteps,),
        in_specs=[
            pl.BlockSpec((tm, y.shape[1]), lambda i: (i, 0)),
            pl.BlockSpec(m.shape, lambda i: (0, 0)),
            pl.BlockSpec(v.shape, lambda i: (0, 0)),
            pl.BlockSpec(g.shape, lambda i: (0, 0)),
            pl.BlockSpec(be.shape, lambda i: (0, 0)),
            pl.BlockSpec(w.shape, lambda i: (0, 0)),
            pl.BlockSpec(b.shape, lambda i: (0, 0)),
        ],
        out_specs=[
            pl.BlockSpec((tm, h), lambda i: (i, 0)),
            pl.BlockSpec((1, h), lambda i: (0, 0)),
        ],
        out_shape=[
            jax.ShapeDtypeStruct((n, h), jnp.float32),
            jax.ShapeDtypeStruct((1, h), jnp.float32),
        ],
        scratch_shapes=[pltpu.VMEM((1, h), jnp.float32)],
    )(y, m, v, g, be, w, b)


def _bn_relu(z, m, v, g, be, tm, out_dtype=jnp.float32):
    n, h = z.shape
    nsteps = n // tm
    return pl.pallas_call(
        _bn_relu_kernel,
        grid=(nsteps,),
        in_specs=[
            pl.BlockSpec((tm, h), lambda i: (i, 0)),
            pl.BlockSpec(m.shape, lambda i: (0, 0)),
            pl.BlockSpec(v.shape, lambda i: (0, 0)),
            pl.BlockSpec(g.shape, lambda i: (0, 0)),
            pl.BlockSpec(be.shape, lambda i: (0, 0)),
        ],
        out_specs=pl.BlockSpec((tm, h), lambda i: (i, 0)),
        out_shape=jax.ShapeDtypeStruct((n, h), out_dtype),
    )(z, m, v, g, be)


def kernel(Adj, feats, W0_1, b0_1, g0_1, be0_1, W0_2, b0_2, bn0_g, bn0_b,
           W1_1, b1_1, g1_1, be1_1, W1_2, b1_2, bn1_g, bn1_b):
    n = Adj.shape[0]
    tm_big = _pick_tile(n, 400)
    tm_small = _pick_tile(n, 2048)
    kc = n

    x0 = feats.astype(jnp.bfloat16)
    r2 = lambda a: a.reshape(1, -1)

    y0, s1 = _pool_mm(Adj, x0, W0_1, r2(b0_1), tm_big, kc)
    m, v = _mean_var(y0, s1, n, tm_small)
    z0, s1 = _bn_mm(y0, m, v, r2(g0_1), r2(be0_1), W0_2, r2(b0_2), tm_small)
    m, v = _mean_var(z0, s1, n, tm_small)
    h0 = _bn_relu(z0, m, v, r2(bn0_g), r2(bn0_b), tm_small,
                  out_dtype=jnp.bfloat16)

    y1, s1 = _pool_mm(Adj, h0, W1_1, r2(b1_1), tm_big, kc)
    m, v = _mean_var(y1, s1, n, tm_small)
    z1, s1 = _bn_mm(y1, m, v, r2(g1_1), r2(be1_1), W1_2, r2(b1_2), tm_small)
    m, v = _mean_var(z1, s1, n, tm_small)

    return _bn_relu(z1, m, v, r2(bn1_g), r2(bn1_b), tm_small)
